# baseline (speedup 1.0000x reference)
import jax
import jax.numpy as jnp
from jax.experimental import pallas as pl

_ALL_ANCHORS = [(12, 16), (19, 36), (40, 28), (36, 75), (76, 55),
                (72, 146), (142, 110), (192, 243), (459, 401)]
_ANCHOR_MASK = [6, 7, 8]
_N_ATTRS = 85
_N_ANCHORS = 3


def _yolo_body(stride_ref, x_ref, o_ref):
    v = x_ref[0]
    stride = stride_ref[0, 0]

    sig = jax.nn.sigmoid(v)
    ex = jnp.exp(v)

    k = jax.lax.broadcasted_iota(jnp.int32, v.shape, 0)
    col = jax.lax.broadcasted_iota(jnp.int32, v.shape, 1)
    c = k % _N_ATTRS
    gx = (col % 19).astype(jnp.float32)
    gy = (col // 19).astype(jnp.float32)

    aw_tab = [float(_ALL_ANCHORS[i][0]) for i in _ANCHOR_MASK]
    ah_tab = [float(_ALL_ANCHORS[i][1]) for i in _ANCHOR_MASK]
    aw = jnp.where(k < _N_ATTRS, aw_tab[0],
                   jnp.where(k < 2 * _N_ATTRS, aw_tab[1], aw_tab[2]))
    ah = jnp.where(k < _N_ATTRS, ah_tab[0],
                   jnp.where(k < 2 * _N_ATTRS, ah_tab[1], ah_tab[2]))

    out = jnp.where(c == 0, (sig + gx) * stride,
          jnp.where(c == 1, (sig + gy) * stride,
          jnp.where(c == 2, ex * aw,
          jnp.where(c == 3, ex * ah, sig))))
    t = out.T
    for a in range(_N_ANCHORS):
        o_ref[0, a::_N_ANCHORS, :] = t[:, a * _N_ATTRS:(a + 1) * _N_ATTRS]


def kernel(x, input_dim):
    b, ch, h, w = x.shape
    hw = h * w
    xr = x.reshape(b, ch, hw)
    stride = jnp.floor(jnp.asarray(input_dim, jnp.float32) / jnp.float32(h))
    stride = stride.reshape(1, 1)

    out = pl.pallas_call(
        _yolo_body,
        grid=(b,),
        in_specs=[
            pl.BlockSpec((1, 1), lambda i: (0, 0)),
            pl.BlockSpec((1, ch, hw), lambda i: (i, 0, 0)),
        ],
        out_specs=pl.BlockSpec((1, hw * _N_ANCHORS, _N_ATTRS),
                               lambda i: (i, 0, 0)),
        out_shape=jax.ShapeDtypeStruct((b, hw * _N_ANCHORS, _N_ATTRS),
                                       jnp.float32),
    )(stride, xr)
    return out


# speedup vs baseline: 1.1989x; 1.1989x over previous
import jax
import jax.numpy as jnp
from jax.experimental import pallas as pl

_ALL_ANCHORS = [(12, 16), (19, 36), (40, 28), (36, 75), (76, 55),
                (72, 146), (142, 110), (192, 243), (459, 401)]
_ANCHOR_MASK = [6, 7, 8]
_N_ATTRS = 85
_N_ANCHORS = 3


_BLK = 4


def _yolo_body(stride_ref, x_ref, o_ref):
    stride = stride_ref[0, 0]
    for img in range(_BLK):
        v = x_ref[img]

        sig = jax.nn.sigmoid(v)
        ex = jnp.exp(v)

        k = jax.lax.broadcasted_iota(jnp.int32, v.shape, 0)
        col = jax.lax.broadcasted_iota(jnp.int32, v.shape, 1)
        c = k % _N_ATTRS
        gx = (col % 19).astype(jnp.float32)
        gy = (col // 19).astype(jnp.float32)

        aw_tab = [float(_ALL_ANCHORS[i][0]) for i in _ANCHOR_MASK]
        ah_tab = [float(_ALL_ANCHORS[i][1]) for i in _ANCHOR_MASK]
        aw = jnp.where(k < _N_ATTRS, aw_tab[0],
                       jnp.where(k < 2 * _N_ATTRS, aw_tab[1], aw_tab[2]))
        ah = jnp.where(k < _N_ATTRS, ah_tab[0],
                       jnp.where(k < 2 * _N_ATTRS, ah_tab[1], ah_tab[2]))

        out = jnp.where(c == 0, (sig + gx) * stride,
              jnp.where(c == 1, (sig + gy) * stride,
              jnp.where(c == 2, ex * aw,
              jnp.where(c == 3, ex * ah, sig))))
        t = out.T
        for a in range(_N_ANCHORS):
            o_ref[img, a::_N_ANCHORS, :] = t[:, a * _N_ATTRS:(a + 1) * _N_ATTRS]


def kernel(x, input_dim):
    b, ch, h, w = x.shape
    hw = h * w
    xr = x.reshape(b, ch, hw)
    stride = jnp.floor(jnp.asarray(input_dim, jnp.float32) / jnp.float32(h))
    stride = stride.reshape(1, 1)

    out = pl.pallas_call(
        _yolo_body,
        grid=(b // _BLK,),
        in_specs=[
            pl.BlockSpec((1, 1), lambda i: (0, 0)),
            pl.BlockSpec((_BLK, ch, hw), lambda i: (i, 0, 0)),
        ],
        out_specs=pl.BlockSpec((_BLK, hw * _N_ANCHORS, _N_ATTRS),
                               lambda i: (i, 0, 0)),
        out_shape=jax.ShapeDtypeStruct((b, hw * _N_ANCHORS, _N_ATTRS),
                                       jnp.float32),
    )(stride, xr)
    return out
